# baseline (speedup 1.0000x reference)
import jax
import jax.numpy as jnp
from jax import lax
from jax.experimental import pallas as pl
from jax.experimental.pallas import tpu as pltpu
from jax.experimental.pallas import tpu_sc as plsc

N = 100000
E = 3200000
NC = 2
NS = 16
NW = NC * NS
BLK_E = 512
NBLK = 196
EW = NBLK * BLK_E
E_PAD = NW * EW
ACC_R = 100096
ZR = ACC_R // NS
ROWS128 = ACC_R // 8


def _edge_loop(load_fire, wait_gathers, scatter_drain):
    load_fire(0, 0)

    def pair(p, carry):
        b0 = 2 * p
        load_fire(b0 + 1, 1)
        wait_gathers(0)
        scatter_drain(0)
        load_fire(b0 + 2, 0)
        wait_gathers(1)
        scatter_drain(1)
        return carry

    lax.fori_loop(0, NBLK // 2, pair, 0)
    wait_gathers(0)


def _sc_agg8_body(t_hbm, s_hbm, d_hbm, z8_hbm, zc_hbm, ones_hbm,
                  out8_hbm, outc_hbm,
                  sA, dA, rA, sB, dB, rB, ones_v, acc8, accc,
                  gsA, gsB, ssA, ssB):
    cid = lax.axis_index("c")
    sid = lax.axis_index("s")
    wid = sid * NC + cid

    pltpu.sync_copy(z8_hbm, acc8.at[pl.ds(sid * ZR, ZR)])
    pltpu.sync_copy(zc_hbm, accc.at[pl.ds(sid * ZR, ZR)])
    pltpu.sync_copy(ones_hbm, ones_v)
    plsc.subcore_barrier()

    ebase = wid * ((NBLK + 1) * BLK_E)
    bufs = ((sA, dA, rA, gsA, ssA), (sB, dB, rB, gsB, ssB))

    def load_fire(b, i):
        sbuf, dbuf, rows, gs, _ = bufs[i]
        pltpu.sync_copy(s_hbm.at[pl.ds(ebase + b * BLK_E, BLK_E)], sbuf)
        pltpu.sync_copy(d_hbm.at[pl.ds(ebase + b * BLK_E, BLK_E)], dbuf)
        pltpu.async_copy(t_hbm.at[sbuf], rows, gs)

    def wait_gathers(i):
        sbuf, _, rows, gs, _ = bufs[i]
        pltpu.make_async_copy(t_hbm.at[sbuf], rows, gs).wait()

    def scatter_drain(i):
        _, dbuf, rows, _, ss = bufs[i]
        pltpu.async_copy(rows, acc8.at[dbuf], ss, add=True)
        pltpu.async_copy(ones_v, accc.at[dbuf], ss, add=True)
        pltpu.make_async_copy(rows, acc8.at[dbuf], ss).wait()
        pltpu.make_async_copy(ones_v, accc.at[dbuf], ss).wait()

    _edge_loop(load_fire, wait_gathers, scatter_drain)

    plsc.subcore_barrier()
    pltpu.sync_copy(acc8.at[pl.ds(sid * ZR, ZR)],
                    out8_hbm.at[cid].at[pl.ds(sid * ZR, ZR)])
    pltpu.sync_copy(accc.at[pl.ds(sid * ZR, ZR)],
                    outc_hbm.at[cid].at[pl.ds(sid * ZR, ZR)])


def _sc_agg16_body(t_hbm, s_hbm, d_hbm, out_hbm,
                   sA, dA, rA, sB, dB, rB, acc, gsA, gsB, ssA, ssB):
    cid = lax.axis_index("c")
    sid = lax.axis_index("s")
    wid = sid * NC + cid

    def zrow(i, carry):
        rA[i] = jnp.zeros((16,), jnp.float32)
        return carry
    lax.fori_loop(0, BLK_E, zrow, 0)
    base = sid * ZR
    for k in range(ZR // BLK_E):
        pltpu.sync_copy(rA, acc.at[pl.ds(base + k * BLK_E, BLK_E)])
    rem = ZR % BLK_E
    if rem:
        pltpu.sync_copy(rA.at[pl.ds(0, rem)],
                        acc.at[pl.ds(base + (ZR // BLK_E) * BLK_E, rem)])
    plsc.subcore_barrier()

    ebase = wid * ((NBLK + 1) * BLK_E)
    bufs = ((sA, dA, rA, gsA, ssA), (sB, dB, rB, gsB, ssB))

    def load_fire(b, i):
        sbuf, dbuf, rows, gs, _ = bufs[i]
        pltpu.sync_copy(s_hbm.at[pl.ds(ebase + b * BLK_E, BLK_E)], sbuf)
        pltpu.sync_copy(d_hbm.at[pl.ds(ebase + b * BLK_E, BLK_E)], dbuf)
        pltpu.async_copy(t_hbm.at[sbuf], rows, gs)

    def wait_gathers(i):
        sbuf, _, rows, gs, _ = bufs[i]
        pltpu.make_async_copy(t_hbm.at[sbuf], rows, gs).wait()

    def scatter_drain(i):
        _, dbuf, rows, _, ss = bufs[i]
        pltpu.async_copy(rows, acc.at[dbuf], ss, add=True)
        pltpu.make_async_copy(rows, acc.at[dbuf], ss).wait()

    _edge_loop(load_fire, wait_gathers, scatter_drain)

    plsc.subcore_barrier()
    pltpu.sync_copy(acc.at[pl.ds(sid * ZR, ZR)],
                    out_hbm.at[cid].at[pl.ds(sid * ZR, ZR)])


_MESH = dict(mesh=plsc.VectorSubcoreMesh(core_axis_name="c", subcore_axis_name="s"),
             compiler_params=pltpu.CompilerParams(use_tc_tiling_on_sc=False))


def _sc_agg8(table8, src_hbm, dst_hbm, z8, zc, ones):
    f = pl.kernel(
        _sc_agg8_body,
        out_type=(jax.ShapeDtypeStruct((NC, ACC_R, 8), jnp.float32),
                  jax.ShapeDtypeStruct((NC, ACC_R, 1), jnp.float32)),
        scratch_types=[
            pltpu.VMEM((BLK_E,), jnp.int32),
            pltpu.VMEM((BLK_E,), jnp.int32),
            pltpu.VMEM((BLK_E, 8), jnp.float32),
            pltpu.VMEM((BLK_E,), jnp.int32),
            pltpu.VMEM((BLK_E,), jnp.int32),
            pltpu.VMEM((BLK_E, 8), jnp.float32),
            pltpu.VMEM((BLK_E, 1), jnp.float32),
            pltpu.VMEM_SHARED((ACC_R, 8), jnp.float32),
            pltpu.VMEM_SHARED((ACC_R, 1), jnp.float32),
            pltpu.SemaphoreType.DMA,
            pltpu.SemaphoreType.DMA,
            pltpu.SemaphoreType.DMA,
            pltpu.SemaphoreType.DMA,
        ],
        **_MESH,
    )
    return f(table8, src_hbm, dst_hbm, z8, zc, ones)


def _sc_agg16(table, src_hbm, dst_hbm):
    f = pl.kernel(
        _sc_agg16_body,
        out_type=jax.ShapeDtypeStruct((NC, ACC_R, 16), jnp.float32),
        scratch_types=[
            pltpu.VMEM((BLK_E,), jnp.int32),
            pltpu.VMEM((BLK_E,), jnp.int32),
            pltpu.VMEM((BLK_E, 16), jnp.float32),
            pltpu.VMEM((BLK_E,), jnp.int32),
            pltpu.VMEM((BLK_E,), jnp.int32),
            pltpu.VMEM((BLK_E, 16), jnp.float32),
            pltpu.VMEM_SHARED((ACC_R, 16), jnp.float32),
            pltpu.SemaphoreType.DMA,
            pltpu.SemaphoreType.DMA,
            pltpu.SemaphoreType.DMA,
            pltpu.SemaphoreType.DMA,
        ],
        **_MESH,
    )
    return f(table, src_hbm, dst_hbm)


_G = 2
_BN16 = ROWS128 // _G
_BN8 = (ACC_R // 16) // _G


def _tc_layer1_body(p_ref, t_ref, c_ref, al_ref, ah_ref, bl_ref, bh_ref,
                    b_ref, h_ref):
    s = p_ref[0] + p_ref[1]
    t = t_ref[...]
    sw = jnp.stack(
        [jnp.dot(s, al_ref[...], preferred_element_type=jnp.float32),
         jnp.dot(s, ah_ref[...], preferred_element_type=jnp.float32)],
        axis=1).reshape(_BN16, 128)
    tw = jnp.stack(
        [jnp.dot(t, bl_ref[...], preferred_element_type=jnp.float32),
         jnp.dot(t, bh_ref[...], preferred_element_type=jnp.float32)],
        axis=1).reshape(_BN16, 128)
    mean_w = sw / jnp.maximum(c_ref[...], 1.0)
    h_ref[...] = jnp.maximum(mean_w + tw + b_ref[...], 0.0)


def _tc_layer2_body(p2_ref, h_ref, c_ref, wl_ref, wr_ref, b_ref, o_ref):
    s2 = p2_ref[0] + p2_ref[1]
    sw = jnp.dot(s2, wl_ref[...], preferred_element_type=jnp.float32)
    o_ref[...] = (sw / jnp.maximum(c_ref[...], 1.0)
                  + jnp.dot(h_ref[...], wr_ref[...],
                            preferred_element_type=jnp.float32)
                  + b_ref[...])


def _tc_layer1(p8, t8, cntbc, AL, AH, BL, BH, b1t):
    return pl.pallas_call(
        _tc_layer1_body,
        grid=(_G,),
        in_specs=[
            pl.BlockSpec((NC, _BN8, 128), lambda i: (0, i, 0)),
            pl.BlockSpec((_BN8, 128), lambda i: (i, 0)),
            pl.BlockSpec((_BN16, 128), lambda i: (i, 0)),
            pl.BlockSpec((128, 128), lambda i: (0, 0)),
            pl.BlockSpec((128, 128), lambda i: (0, 0)),
            pl.BlockSpec((128, 128), lambda i: (0, 0)),
            pl.BlockSpec((128, 128), lambda i: (0, 0)),
            pl.BlockSpec((1, 128), lambda i: (0, 0)),
        ],
        out_specs=pl.BlockSpec((_BN16, 128), lambda i: (i, 0)),
        out_shape=jax.ShapeDtypeStruct((ROWS128, 128), jnp.float32),
    )(p8, t8, cntbc, AL, AH, BL, BH, b1t)


def _tc_layer2(p2, h, cntbc, WLbd, WRbd, b2t):
    return pl.pallas_call(
        _tc_layer2_body,
        grid=(_G,),
        in_specs=[
            pl.BlockSpec((NC, _BN16, 128), lambda i: (0, i, 0)),
            pl.BlockSpec((_BN16, 128), lambda i: (i, 0)),
            pl.BlockSpec((_BN16, 128), lambda i: (i, 0)),
            pl.BlockSpec((128, 128), lambda i: (0, 0)),
            pl.BlockSpec((128, 128), lambda i: (0, 0)),
            pl.BlockSpec((1, 128), lambda i: (0, 0)),
        ],
        out_specs=pl.BlockSpec((_BN16, 128), lambda i: (i, 0)),
        out_shape=jax.ShapeDtypeStruct((ROWS128, 128), jnp.float32),
    )(p2, h, cntbc, WLbd, WRbd, b2t)


def kernel(x, edge_index, W1_l, b1, W1_r, W2_l, b2, W2_r):
    src = edge_index[0]
    dst = edge_index[1]
    pad = E_PAD - E
    src_p = jnp.concatenate([src, jnp.zeros((pad,), jnp.int32)])
    dst_p = jnp.concatenate([dst, jnp.full((pad,), N, jnp.int32)])
    s3 = jnp.pad(src_p.reshape(NW, NBLK, BLK_E), ((0, 0), (0, 1), (0, 0)))
    d3 = jnp.pad(dst_p.reshape(NW, NBLK, BLK_E), ((0, 0), (0, 1), (0, 0)))
    src_flat = s3.reshape(-1)
    dst_flat = d3.reshape(-1)

    t8_128 = jnp.pad(x.reshape(N // 16, 128), ((0, (ACC_R - N) // 16), (0, 0)))

    z8 = jnp.zeros((ZR, 8), jnp.float32)
    zc = jnp.zeros((ZR, 1), jnp.float32)
    ones = jnp.ones((BLK_E, 1), jnp.float32)

    M_lo = jnp.zeros((16, 8), jnp.float32).at[:8, :].set(jnp.eye(8))
    M_hi = jnp.zeros((16, 8), jnp.float32).at[8:, :].set(jnp.eye(8))
    AL = jnp.kron(M_lo, W1_l)
    AH = jnp.kron(M_hi, W1_l)
    BL = jnp.kron(M_lo, W1_r)
    BH = jnp.kron(M_hi, W1_r)
    eye8 = jnp.eye(8, dtype=jnp.float32)
    WL2 = jnp.kron(eye8, W2_l)
    WR2 = jnp.kron(eye8, W2_r)
    b1t = jnp.tile(b1, 8).reshape(1, 128)
    b2t = jnp.tile(b2, 8).reshape(1, 128)

    p8, pc = _sc_agg8(t8_128.reshape(ACC_R, 8), src_flat, dst_flat,
                      z8, zc, ones)
    cnt = pc[0, :, 0] + pc[1, :, 0]
    cntbc = jnp.repeat(cnt[:, None], 16, axis=1).reshape(ROWS128, 128)

    h128 = _tc_layer1(p8.reshape(NC, ACC_R // 16, 128),
                      t8_128, cntbc, AL, AH, BL, BH, b1t)
    p2 = _sc_agg16(h128.reshape(ACC_R, 16), src_flat, dst_flat)
    out128 = _tc_layer2(p2.reshape(NC, ROWS128, 128),
                        h128, cntbc, WL2, WR2, b2t)
    return out128.reshape(ACC_R, 16)[:N]
